# Initial kernel scaffold; baseline (speedup 1.0000x reference)
import jax
import jax.numpy as jnp
from jax.experimental import pallas as pl


def kernel(user_id, item_id, user_emb, item_emb, W0, b0, W1, b1, W2, b2, W3, b3, W4, b4, g0, be0, g1, be1, g2, be2, g3, be3):
    raise NotImplementedError("write your pallas kernel here")


# speedup vs baseline: 1.0398x; 1.0398x over previous
import functools

import jax
import jax.numpy as jnp
from jax import lax
from jax.experimental import pallas as pl
from jax.experimental.pallas import tpu as pltpu
from jax.experimental.pallas import tpu_sc as plsc

_U = 100000
_D = 64


def _gather_body(nc, b_per_w, uidx_hbm, iidx_hbm, utab_hbm, itab_hbm,
                 out_u, out_v, uidx_v, iidx_v, urows_v, irows_v, sem_u, sem_v):
    wid = lax.axis_index("s") * nc + lax.axis_index("c")
    base = wid * b_per_w
    pltpu.sync_copy(uidx_hbm.at[pl.ds(base, b_per_w)], uidx_v)
    pltpu.sync_copy(iidx_hbm.at[pl.ds(base, b_per_w)], iidx_v)
    cu = pltpu.async_copy(utab_hbm.at[uidx_v], urows_v, sem_u)
    ci = pltpu.async_copy(itab_hbm.at[iidx_v], irows_v, sem_v)
    cu.wait()
    ci.wait()
    pltpu.sync_copy(urows_v, out_u.at[pl.ds(base, b_per_w)])
    pltpu.sync_copy(irows_v, out_v.at[pl.ds(base, b_per_w)])


def _sc_gather(uidx, iidx, utab, itab):
    b = uidx.shape[0]
    info = plsc.get_sparse_core_info()
    nc, ns = info.num_cores, info.num_subcores
    nw = nc * ns
    b_per_w = b // nw
    mesh = plsc.VectorSubcoreMesh(core_axis_name="c", subcore_axis_name="s")
    k = pl.kernel(
        functools.partial(_gather_body, nc, b_per_w),
        out_type=[jax.ShapeDtypeStruct((b, _D), jnp.float32),
                  jax.ShapeDtypeStruct((b, _D), jnp.float32)],
        mesh=mesh,
        scratch_types=[
            pltpu.VMEM((b_per_w,), jnp.int32),
            pltpu.VMEM((b_per_w,), jnp.int32),
            pltpu.VMEM((b_per_w, _D), jnp.float32),
            pltpu.VMEM((b_per_w, _D), jnp.float32),
            pltpu.SemaphoreType.DMA,
            pltpu.SemaphoreType.DMA,
        ],
        compiler_params=pltpu.CompilerParams(use_tc_tiling_on_sc=False),
    )
    return k(uidx, iidx, utab, itab)


def _bn_relu(x, g, be):
    m = jnp.mean(x, axis=0, keepdims=True)
    xc = x - m
    var = jnp.mean(xc * xc, axis=0, keepdims=True)
    return jnp.maximum(g * xc * lax.rsqrt(var + 1e-5) + be, 0.0)


def _mlp_body(u_ref, v_ref, w0u_ref, w0v_ref, b0_ref, w1_ref, b1_ref,
              w2_ref, b2_ref, w3_ref, b3_ref, w4_ref, b4_ref,
              g0_ref, be0_ref, g1_ref, be1_ref, g2_ref, be2_ref,
              g3_ref, be3_ref, out_ref):
    f32 = jnp.float32
    x = (jnp.dot(u_ref[...], w0u_ref[...], preferred_element_type=f32)
         + jnp.dot(v_ref[...], w0v_ref[...], preferred_element_type=f32)
         + b0_ref[...])
    x = _bn_relu(x, g0_ref[...], be0_ref[...])
    x = jnp.dot(x, w1_ref[...], preferred_element_type=f32) + b1_ref[...]
    x = _bn_relu(x, g1_ref[...], be1_ref[...])
    x = jnp.dot(x, w2_ref[...], preferred_element_type=f32) + b2_ref[...]
    x = _bn_relu(x, g2_ref[...], be2_ref[...])
    x = jnp.dot(x, w3_ref[...], preferred_element_type=f32) + b3_ref[...]
    x = _bn_relu(x, g3_ref[...], be3_ref[...])
    x = jnp.dot(x, w4_ref[...], preferred_element_type=f32) + b4_ref[...]
    out_ref[...] = 5.0 * jax.nn.sigmoid(x)


def kernel(user_id, item_id, user_emb, item_emb, W0, b0, W1, b1, W2, b2,
           W3, b3, W4, b4, g0, be0, g1, be1, g2, be2, g3, be3):
    b = user_id.shape[0]
    uidx = (user_id % _U).astype(jnp.int32)
    iidx = (item_id % _U).astype(jnp.int32)
    u, v = _sc_gather(uidx, iidx, user_emb, item_emb)

    mlp = pl.pallas_call(
        _mlp_body,
        out_shape=jax.ShapeDtypeStruct((b, 1), jnp.float32),
    )
    return mlp(
        u, v,
        W0[:, :_D].T, W0[:, _D:].T, b0.reshape(1, -1),
        W1.T, b1.reshape(1, -1),
        W2.T, b2.reshape(1, -1),
        W3.T, b3.reshape(1, -1),
        W4.T, b4.reshape(1, -1),
        g0.reshape(1, -1), be0.reshape(1, -1),
        g1.reshape(1, -1), be1.reshape(1, -1),
        g2.reshape(1, -1), be2.reshape(1, -1),
        g3.reshape(1, -1), be3.reshape(1, -1),
    )
